# baseline (speedup 1.0000x reference)
import functools

import jax
import jax.numpy as jnp
from jax import lax
from jax.experimental import pallas as pl
from jax.experimental.pallas import tpu as pltpu
from jax.experimental.pallas import tpu_sc as plsc

NC, NS, L = 2, 16, 16
NW = NC * NS
B = 4
S = 4096
D = 768
SLICES = D // L
POS_PER_W = S // NW
CHUNK = 32
NCHUNK = POS_PER_W // CHUNK
NITER = NCHUNK * B
NSLOT = 4

_mesh = plsc.VectorSubcoreMesh(
    core_axis_name="c", subcore_axis_name="s", num_cores=NC, num_subcores=NS
)

_gdnums = lax.GatherDimensionNumbers(
    offset_dims=(), collapsed_slice_dims=(0,), start_index_map=(0,)
)


@functools.partial(
    pl.kernel,
    out_type=jax.ShapeDtypeStruct((B * S, D), jnp.float32),
    mesh=_mesh,
    scratch_types=[
        pltpu.VMEM((B * POS_PER_W,), jnp.int32),
        [pltpu.VMEM((CHUNK, D), jnp.float32)] * NSLOT,
        [pltpu.VMEM((CHUNK, D), jnp.float32)] * 1,
        [pltpu.SemaphoreType.DMA] * NSLOT,
        [pltpu.SemaphoreType.DMA] * NSLOT,
        [pltpu.SemaphoreType.DMA] * 1,
        pltpu.SemaphoreType.DMA,
    ],
)
def _embed(x_hbm, table_hbm, pos_hbm, out_hbm,
           idx_v, rows, pos, gsem, ssem, psem, isem):
    wid = lax.axis_index("s") * NC + lax.axis_index("c")
    pos0 = wid * POS_PER_W

    icp = [
        pltpu.async_copy(
            x_hbm.at[pl.ds(b * S + pos0, POS_PER_W)],
            idx_v.at[pl.ds(b * POS_PER_W, POS_PER_W)],
            isem,
        )
        for b in range(B)
    ]

    def idx_slice(i):
        c, b = divmod(i, B)
        return pl.ds(b * POS_PER_W + c * CHUNK, CHUNK)

    def gather(i, slot):
        return pltpu.async_copy(
            table_hbm.at[idx_v.at[idx_slice(i)]], rows[slot], gsem[slot]
        )

    pos_cp = [None] * NCHUNK
    pos_cp[0] = pltpu.async_copy(
        pos_hbm.at[pl.ds(pos0, CHUNK)], pos[0], psem[0]
    )
    icp[0].wait()
    gather_cp = [None] * NITER
    gather_cp[0] = gather(0, 0)
    for b in range(1, B):
        icp[b].wait()
    store_cp = [None] * NSLOT

    for i in range(NITER):
        c, b = divmod(i, B)
        slot = i % NSLOT
        if b == 0:
            pos_cp[c].wait()
        gather_cp[i].wait()
        if i + 1 < NITER:
            nslot = (i + 1) % NSLOT
            if store_cp[nslot] is not None:
                store_cp[nslot].wait()
            gather_cp[i + 1] = gather(i + 1, nslot)

        rv = rows[slot]
        pv = pos[0]
        ibase = b * POS_PER_W + c * CHUNK

        @plsc.parallel_loop(0, CHUNK, 1, unroll=2)
        def body(j):
            base = (j // L) * L
            iv = idx_v[pl.ds(ibase + base, L)]
            sv_g = jnp.where(iv == 1, 0.0, 1.0)
            lane = jnp.full((L, 1), j - base, jnp.int32)
            svec = lax.gather(
                sv_g, lane, _gdnums, (1,),
                mode=lax.GatherScatterMode.PROMISE_IN_BOUNDS,
            )
            for k in range(SLICES):
                sl = pl.ds(k * L, L)
                rv[j, sl] = rv[j, sl] * svec + pv[j, sl]

        t0 = b * S + pos0 + c * CHUNK
        store_cp[slot] = pltpu.async_copy(
            rv, out_hbm.at[pl.ds(t0, CHUNK)], ssem[slot]
        )
        if b == B - 1 and c + 1 < NCHUNK:
            pos_cp[c + 1] = pltpu.async_copy(
                pos_hbm.at[pl.ds(pos0 + (c + 1) * CHUNK, CHUNK)],
                pos[0], psem[0],
            )

    for slot in range(NSLOT):
        if store_cp[slot] is not None:
            store_cp[slot].wait()


def kernel(x, table, pos_enc):
    out = _embed(x.reshape(-1), table, pos_enc)
    return out.reshape(B, S, D)


# speedup vs baseline: 1.0299x; 1.0299x over previous
import functools

import jax
import jax.numpy as jnp
from jax import lax
from jax.experimental import pallas as pl
from jax.experimental.pallas import tpu as pltpu
from jax.experimental.pallas import tpu_sc as plsc

NC, NS, L = 2, 16, 16
NW = NC * NS
B = 4
S = 4096
D = 768
SLICES = D // L
POS_PER_W = S // NW
CHUNK = 32
NCHUNK = POS_PER_W // CHUNK
NITER = NCHUNK * B
NSLOT = 3

_mesh = plsc.VectorSubcoreMesh(
    core_axis_name="c", subcore_axis_name="s", num_cores=NC, num_subcores=NS
)

_gdnums = lax.GatherDimensionNumbers(
    offset_dims=(), collapsed_slice_dims=(0,), start_index_map=(0,)
)


@functools.partial(
    pl.kernel,
    out_type=jax.ShapeDtypeStruct((B * S, D), jnp.float32),
    mesh=_mesh,
    scratch_types=[
        pltpu.VMEM((B * POS_PER_W,), jnp.int32),
        [pltpu.VMEM((CHUNK, D), jnp.float32)] * NSLOT,
        [pltpu.VMEM((CHUNK, D), jnp.float32)] * 2,
        [pltpu.SemaphoreType.DMA] * NSLOT,
        [pltpu.SemaphoreType.DMA] * NSLOT,
        [pltpu.SemaphoreType.DMA] * 2,
        pltpu.SemaphoreType.DMA,
    ],
)
def _embed(x_hbm, table_hbm, pos_hbm, out_hbm,
           idx_v, rows, pos, gsem, ssem, psem, isem):
    wid = lax.axis_index("s") * NC + lax.axis_index("c")
    pos0 = wid * POS_PER_W

    icp = [
        pltpu.async_copy(
            x_hbm.at[pl.ds(b * S + pos0, POS_PER_W)],
            idx_v.at[pl.ds(b * POS_PER_W, POS_PER_W)],
            isem,
        )
        for b in range(B)
    ]

    def idx_slice(i):
        c, b = divmod(i, B)
        return pl.ds(b * POS_PER_W + c * CHUNK, CHUNK)

    def gather(i, slot):
        c, b = divmod(i, B)
        s0 = b * POS_PER_W + c * CHUNK
        H = CHUNK // 2
        return [
            pltpu.async_copy(
                table_hbm.at[idx_v.at[pl.ds(s0 + h * H, H)]],
                rows[slot].at[pl.ds(h * H, H)],
                gsem[slot],
            )
            for h in range(2)
        ]

    pos_cp = [None] * NCHUNK
    pos_cp[0] = pltpu.async_copy(
        pos_hbm.at[pl.ds(pos0, CHUNK)], pos[0], psem[0]
    )
    icp[0].wait()
    gather_cp = [None] * NITER
    gather_cp[0] = gather(0, 0)
    for b in range(1, B):
        icp[b].wait()
    store_cp = [None] * NSLOT

    for i in range(NITER):
        c, b = divmod(i, B)
        slot = i % NSLOT
        if b == 0:
            pos_cp[c].wait()
            if c + 1 < NCHUNK:
                pos_cp[c + 1] = pltpu.async_copy(
                    pos_hbm.at[pl.ds(pos0 + (c + 1) * CHUNK, CHUNK)],
                    pos[(c + 1) % 2],
                    psem[(c + 1) % 2],
                )
        for gcp in gather_cp[i]:
            gcp.wait()
        if i + 1 < NITER:
            nslot = (i + 1) % NSLOT
            if store_cp[nslot] is not None:
                store_cp[nslot].wait()
            gather_cp[i + 1] = gather(i + 1, nslot)

        rv = rows[slot]
        pv = pos[c % 2]
        ibase = b * POS_PER_W + c * CHUNK

        @plsc.parallel_loop(0, CHUNK, 1, unroll=2)
        def body(j):
            base = (j // L) * L
            iv = idx_v[pl.ds(ibase + base, L)]
            sv_g = jnp.where(iv == 1, 0.0, 1.0)
            lane = jnp.full((L, 1), j - base, jnp.int32)
            svec = lax.gather(
                sv_g, lane, _gdnums, (1,),
                mode=lax.GatherScatterMode.PROMISE_IN_BOUNDS,
            )
            for k in range(SLICES):
                sl = pl.ds(k * L, L)
                rv[j, sl] = rv[j, sl] * svec + pv[j, sl]

        t0 = b * S + pos0 + c * CHUNK
        store_cp[slot] = pltpu.async_copy(
            rv, out_hbm.at[pl.ds(t0, CHUNK)], ssem[slot]
        )

    for slot in range(NSLOT):
        if store_cp[slot] is not None:
            store_cp[slot].wait()


def kernel(x, table, pos_enc):
    out = _embed(x.reshape(-1), table, pos_enc)
    return out.reshape(B, S, D)
